# baseline (speedup 1.0000x reference)
import functools

import jax
import jax.numpy as jnp
from jax import lax
from jax.experimental import pallas as pl
from jax.experimental.pallas import tpu as pltpu
from jax.experimental.pallas import tpu_sc as plsc

_NC = 2
_NS = 16
_NW = _NC * _NS

_B = 16384
_D = 64
_B_PER_W = _B // _NW


def _sc_gather_body(cid_hbm, did_hbm, ctab_hbm, dtab_hbm, out_c_hbm, out_d_hbm,
                    idx_c, rows_c, idx_d, rows_d, sem_c, sem_d):
  wid = lax.axis_index("s") * _NC + lax.axis_index("c")
  base = wid * _B_PER_W
  pltpu.sync_copy(cid_hbm.at[pl.ds(base, _B_PER_W)], idx_c)
  pltpu.sync_copy(did_hbm.at[pl.ds(base, _B_PER_W)], idx_d)
  cp_c = pltpu.async_copy(ctab_hbm.at[idx_c], rows_c, sem_c)
  cp_d = pltpu.async_copy(dtab_hbm.at[idx_d], rows_d, sem_d)
  cp_c.wait()
  cp_d.wait()
  pltpu.sync_copy(rows_c, out_c_hbm.at[pl.ds(base, _B_PER_W)])
  pltpu.sync_copy(rows_d, out_d_hbm.at[pl.ds(base, _B_PER_W)])


def _sc_gather(client_ids, cleaner_ids, client_table, cleaner_table):
  mesh = plsc.VectorSubcoreMesh(core_axis_name="c", subcore_axis_name="s")
  fn = pl.kernel(
      _sc_gather_body,
      out_type=[
          jax.ShapeDtypeStruct((_B, _D), jnp.float32),
          jax.ShapeDtypeStruct((_B, _D), jnp.float32),
      ],
      mesh=mesh,
      scratch_types=[
          pltpu.VMEM((_B_PER_W,), jnp.int32),
          pltpu.VMEM((_B_PER_W, _D), jnp.float32),
          pltpu.VMEM((_B_PER_W,), jnp.int32),
          pltpu.VMEM((_B_PER_W, _D), jnp.float32),
          pltpu.SemaphoreType.DMA,
          pltpu.SemaphoreType.DMA,
      ],
      compiler_params=pltpu.CompilerParams(use_tc_tiling_on_sc=False),
  )
  return fn(client_ids, cleaner_ids, client_table, cleaner_table)


_MLP_BLK = 2048


def _mlp_body(ce_ref, cl_ref, w1a_ref, w1b_ref, b1_ref, w2_ref, b2_ref,
              w3_ref, b3_ref, w4_ref, b4_ref, out_ref):
  x = ce_ref[...] @ w1a_ref[...] + cl_ref[...] @ w1b_ref[...] + b1_ref[...]
  h = jnp.maximum(x, 0.0)
  h = jnp.maximum(h @ w2_ref[...] + b2_ref[...], 0.0)
  h = jnp.maximum(h @ w3_ref[...] + b3_ref[...], 0.0)
  out_ref[...] = h @ w4_ref[...] + b4_ref[...]


def _mlp(ce, cl, W1, b1, W2, b2, W3, b3, W4, b4):
  grid = (_B // _MLP_BLK,)
  full = lambda shape: pl.BlockSpec(shape, lambda i: (0, 0))
  return pl.pallas_call(
      _mlp_body,
      grid=grid,
      in_specs=[
          pl.BlockSpec((_MLP_BLK, _D), lambda i: (i, 0)),
          pl.BlockSpec((_MLP_BLK, _D), lambda i: (i, 0)),
          full((_D, 128)),
          full((_D, 128)),
          full((1, 128)),
          full((128, 64)),
          full((1, 64)),
          full((64, 32)),
          full((1, 32)),
          full((32, 1)),
          full((1, 1)),
      ],
      out_specs=pl.BlockSpec((_MLP_BLK, 1), lambda i: (i, 0)),
      out_shape=jax.ShapeDtypeStruct((_B, 1), jnp.float32),
  )(ce, cl, W1[:_D], W1[_D:], b1.reshape(1, -1), W2, b2.reshape(1, -1),
    W3, b3.reshape(1, -1), W4, b4.reshape(1, 1))


@jax.jit
def kernel(client_ids, cleaner_ids, client_table, cleaner_table,
           W1, b1, W2, b2, W3, b3, W4, b4):
  ce, cl = _sc_gather(client_ids.astype(jnp.int32),
                      cleaner_ids.astype(jnp.int32),
                      client_table, cleaner_table)
  out = _mlp(ce, cl, W1, b1, W2, b2, W3, b3, W4, b4)
  return jnp.squeeze(out, axis=-1)
